# baseline (speedup 1.0000x reference)
import functools

import jax
import jax.numpy as jnp
from jax.experimental import pallas as pl
from jax.experimental.pallas import tpu as pltpu

N = 10080
HEADS = 2


def _gat_conv(x, src, dst, W, a_src, a_dst, b, heads, out_ch, concat):
    n = x.shape[0]
    h = (x @ W).reshape(n, heads, out_ch)
    alpha_src = jnp.sum(h * a_src, axis=-1)
    alpha_dst = jnp.sum(h * a_dst, axis=-1)
    alpha = jax.nn.leaky_relu(alpha_src[src] + alpha_dst[dst], negative_slope=0.2)
    amax = jax.ops.segment_max(alpha, dst, num_segments=n)
    ex = jnp.exp(alpha - amax[dst])
    den = jax.ops.segment_sum(ex, dst, num_segments=n)
    att = ex / (den[dst] + 1e-16)
    out = jax.ops.segment_sum(h[src] * att[:, :, None], dst, num_segments=n)
    if concat:
        out = out.reshape(n, heads * out_ch)
    else:
        out = out.mean(axis=1)
    return out + b


def _mlp_body(h_ref, fw1_ref, fb1_ref, fw2_ref, fb2_ref, fw3_ref, fb3_ref,
              fw4_ref, fb4_ref, o_ref):
    h = h_ref[...]
    p = jnp.sum(h.reshape(24, 420, 512), axis=1)
    p = jax.nn.relu(jnp.dot(p, fw1_ref[...], preferred_element_type=jnp.float32)
                    + fb1_ref[...])
    p = jax.nn.relu(jnp.dot(p, fw2_ref[...], preferred_element_type=jnp.float32)
                    + fb2_ref[...])
    p = jax.nn.relu(jnp.dot(p, fw3_ref[...], preferred_element_type=jnp.float32)
                    + fb3_ref[...])
    p = jnp.dot(p, fw4_ref[...], preferred_element_type=jnp.float32) + fb4_ref[...]
    o_ref[...] = p


def _pool_mlp(h, fw1, fb1, fw2, fb2, fw3, fb3, fw4, fb4):
    return pl.pallas_call(
        _mlp_body,
        out_shape=jax.ShapeDtypeStruct((24, 32), jnp.float32),
    )(h, fw1, fb1, fw2, fb2, fw3, fb3, fw4, fb4)


def kernel(x, edge_index, W1, a1_src, a1_dst, b1, W2, a2_src, a2_dst, b2,
           fw1, fb1, fw2, fb2, fw3, fb3, fw4, fb4):
    loops = jnp.arange(N, dtype=edge_index.dtype)
    src = jnp.concatenate([edge_index[0], loops])
    dst = jnp.concatenate([edge_index[1], loops])
    h = jax.nn.relu(_gat_conv(x, src, dst, W1, a1_src, a1_dst, b1, HEADS, 1024, True))
    h = jax.nn.relu(_gat_conv(h, src, dst, W2, a2_src, a2_dst, b2, 1, 512, False))
    return _pool_mlp(h, fw1, fb1, fw2, fb2, fw3, fb3, fw4, fb4)


# speedup vs baseline: 5.6351x; 5.6351x over previous
import functools

import jax
import jax.numpy as jnp
from jax import lax
from jax.experimental import pallas as pl
from jax.experimental.pallas import tpu as pltpu
from jax.experimental.pallas import tpu_sc as plsc

N = 10080
E = 161280
ET = E + N
NTILES = 32
NPT = N // NTILES
EC = 64
EP = ((ET + EC - 1) // EC) * EC
GB = 8
NEG_SLOPE = 0.2
F32 = jnp.float32
I32 = jnp.int32


def _lanes(v):
    return jnp.zeros((16,), I32) + v


def _gat_edge_sc(h, asrc, adst, gmat, srcs, dsts, bounds, bias, *, C, H):
    CH = C // H
    NCH = CH // 16

    mesh = plsc.VectorSubcoreMesh(core_axis_name="c", subcore_axis_name="s")

    def body(h_hbm, asrc_hbm, adst_hbm, g_hbm, srcs_hbm, dsts_hbm, bounds_hbm,
             bias_hbm, out_hbm,
             asrc_v, adst_v, g_v, bias_v, bounds_v, srcs_buf, dsts_buf,
             rows, acc, stage, gs0, gs1, osem):
        wid = lax.axis_index("s") * 2 + lax.axis_index("c")
        n0 = wid * NPT

        pltpu.sync_copy(asrc_hbm, asrc_v)
        pltpu.sync_copy(adst_hbm, adst_v)
        pltpu.sync_copy(g_hbm, g_v)
        pltpu.sync_copy(bias_hbm, bias_v)
        pltpu.sync_copy(bounds_hbm, bounds_v)

        g = [g_v[hh] for hh in range(H)]
        zero16 = jnp.zeros((16,), F32)

        def zeroacc(c2, _):
            acc[pl.ds(c2 * 16, 16)] = zero16
            return 0
        lax.fori_loop(0, C // 16, zeroacc, 0)

        e0 = jnp.max(plsc.load_gather(bounds_v, [_lanes(0) + wid]))
        e1 = jnp.max(plsc.load_gather(bounds_v, [_lanes(1) + wid]))

        pltpu.async_copy(stage, out_hbm.at[n0], osem)

        gsems = (gs0, gs1)

        def issue(bb, slot):
            pltpu.async_copy(
                h_hbm.at[srcs_buf.at[pl.ds(bb * GB, GB)]], rows.at[slot],
                gsems[slot])

        def flush(cn, den):
            pltpu.make_async_copy(stage, out_hbm.at[cn], osem).wait()
            for hh in range(H):
                rden = 1.0 / (den[hh] + 1e-16)
                def norm(c2, _, hh=hh, rden=rden):
                    sl = pl.ds(hh * CH + c2 * 16, 16)
                    stage[sl] = jnp.maximum(acc[sl] * rden + bias_v[sl], 0.0)
                    acc[sl] = zero16
                    return 0
                lax.fori_loop(0, NCH, norm, 0, unroll=4)
            pltpu.async_copy(stage, out_hbm.at[cn], osem)

        def edge(le, slot, j, base, carry):
            e = base + le
            valid = jnp.logical_and(e >= e0, e < e1)

            def do(cn, *den):
                dst_bc = plsc.load_gather(dsts_buf, [_lanes(le)])
                dcur = jnp.max(dst_bc)

                def do_flush(cn, *den):
                    flush(cn, den)
                    return (dcur,) + tuple(zero16 for _ in range(H))

                def keep(cn, *den):
                    return (cn,) + tuple(den)

                cnd = lax.cond(dcur != cn, do_flush, keep, cn, *den)
                cn, den = cnd[0], list(cnd[1:])

                src_bc = plsc.load_gather(srcs_buf, [_lanes(le)])
                for hh in range(H):
                    as_h = plsc.load_gather(asrc_v, [src_bc * H + hh])
                    ad_h = plsc.load_gather(adst_v, [dst_bc * H + hh])
                    s = as_h + ad_h
                    al = jnp.where(s < 0.0, NEG_SLOPE * s, s)
                    t = g[hh] + ad_h
                    cb = jnp.where(t < 0.0, NEG_SLOPE * t, t)
                    ex = jnp.exp(al - cb)
                    den[hh] = den[hh] + ex

                    def mac(c2, _, hh=hh, ex=ex):
                        sl = pl.ds(hh * CH + c2 * 16, 16)
                        plsc.addupdate(acc.at[sl], ex * rows[slot, j, sl])
                        return 0
                    lax.fori_loop(0, NCH, mac, 0, unroll=8)
                return (cn,) + tuple(den)

            def skip(cn, *den):
                return (cn,) + tuple(den)

            out = lax.cond(valid, do, skip, carry[0], *carry[1:])
            return (out[0],) + tuple(out[1:])

        def chunk(ci, carry):
            base = ci * EC
            pltpu.sync_copy(srcs_hbm.at[pl.ds(base, EC)], srcs_buf)
            pltpu.sync_copy(dsts_hbm.at[pl.ds(base, EC)], dsts_buf)
            issue(0, 0)
            for bb in range(EC // GB):
                slot = bb % 2
                if bb + 1 < EC // GB:
                    issue(bb + 1, (bb + 1) % 2)
                pltpu.make_async_copy(
                    h_hbm.at[srcs_buf.at[pl.ds(bb * GB, GB)]], rows.at[slot],
                    gsems[slot]).wait()

                def ebody(j, c, bb=bb, slot=slot):
                    return edge(bb * GB + j, slot, j, base, c)
                carry = lax.fori_loop(0, GB, ebody, carry)
            return carry

        c_lo = e0 // EC
        c_hi = (e1 + EC - 1) // EC
        init = (n0,) + tuple(jnp.zeros((16,), F32) for _ in range(H))
        fin = lax.fori_loop(c_lo, c_hi, chunk, init)
        flush(fin[0], list(fin[1:]))
        pltpu.make_async_copy(stage, out_hbm.at[n0], osem).wait()

    f = pl.kernel(
        body,
        out_type=jax.ShapeDtypeStruct((N, C), F32),
        mesh=mesh,
        compiler_params=pltpu.CompilerParams(needs_layout_passes=False),
        scratch_types=[
            pltpu.VMEM((N * H,), F32),
            pltpu.VMEM((N * H,), F32),
            pltpu.VMEM((H, 16), F32),
            pltpu.VMEM((C,), F32),
            pltpu.VMEM((48,), I32),
            pltpu.VMEM((EC,), I32),
            pltpu.VMEM((EC,), I32),
            pltpu.VMEM((2, GB, C), F32),
            pltpu.VMEM((C,), F32),
            pltpu.VMEM((C,), F32),
            pltpu.SemaphoreType.DMA,
            pltpu.SemaphoreType.DMA,
            pltpu.SemaphoreType.DMA,
        ],
    )
    return f(h, asrc, adst, gmat, srcs, dsts, bounds, bias)


def _prep_edges(edge_index):
    loops = jnp.arange(N, dtype=jnp.int32)
    src = jnp.concatenate([edge_index[0], loops])
    dst = jnp.concatenate([edge_index[1], loops])
    dst_s, src_s = jax.lax.sort_key_val(dst, src)
    tgt = jnp.arange(NTILES + 1, dtype=I32) * NPT
    bounds = jnp.searchsorted(dst_s, tgt, side="left").astype(I32)
    bounds = jnp.minimum(bounds, ET)
    bounds = jnp.concatenate([bounds, jnp.full((48 - NTILES - 1,), ET, I32)])
    src_p = jnp.concatenate([src_s, jnp.zeros((EP - ET,), I32)])
    dst_p = jnp.concatenate([dst_s, jnp.full((EP - ET,), N - 1, I32)])
    return src_p, dst_p, bounds


def _mlp_body(h_ref, fw1_ref, fb1_ref, fw2_ref, fb2_ref, fw3_ref, fb3_ref,
              fw4_ref, fb4_ref, o_ref):
    h = h_ref[...]
    p = jnp.sum(h.reshape(24, 420, 512), axis=1)
    p = jax.nn.relu(jnp.dot(p, fw1_ref[...], preferred_element_type=F32)
                    + fb1_ref[...])
    p = jax.nn.relu(jnp.dot(p, fw2_ref[...], preferred_element_type=F32)
                    + fb2_ref[...])
    p = jax.nn.relu(jnp.dot(p, fw3_ref[...], preferred_element_type=F32)
                    + fb3_ref[...])
    p = jnp.dot(p, fw4_ref[...], preferred_element_type=F32) + fb4_ref[...]
    o_ref[...] = p


def _pool_mlp(h, fw1, fb1, fw2, fb2, fw3, fb3, fw4, fb4):
    return pl.pallas_call(
        _mlp_body,
        out_shape=jax.ShapeDtypeStruct((24, 32), F32),
    )(h, fw1, fb1, fw2, fb2, fw3, fb3, fw4, fb4)


def _gat_layer(x, W, a_src, a_dst, b, srcs, dsts, bounds, *, heads, out_ch):
    h = x @ W
    hh = h.reshape(N, heads, out_ch)
    asrc = jnp.sum(hh * a_src, axis=-1)
    adst = jnp.sum(hh * a_dst, axis=-1)
    g = jnp.max(asrc, axis=0)
    gmat = jnp.broadcast_to(g[:, None], (heads, 16)).astype(F32)
    return _gat_edge_sc(h, asrc.reshape(-1), adst.reshape(-1), gmat,
                        srcs, dsts, bounds, b, C=heads * out_ch, H=heads)


def kernel(x, edge_index, W1, a1_src, a1_dst, b1, W2, a2_src, a2_dst, b2,
           fw1, fb1, fw2, fb2, fw3, fb3, fw4, fb4):
    srcs, dsts, bounds = _prep_edges(edge_index)
    h = _gat_layer(x, W1, a1_src, a1_dst, b1, srcs, dsts, bounds,
                   heads=2, out_ch=1024)
    h = _gat_layer(h, W2, a2_src, a2_dst, b2, srcs, dsts, bounds,
                   heads=1, out_ch=512)
    return _pool_mlp(h, fw1, fb1, fw2, fb2, fw3, fb3, fw4, fb4)


# speedup vs baseline: 12.1401x; 2.1544x over previous
import functools

import jax
import jax.numpy as jnp
from jax import lax
from jax.experimental import pallas as pl
from jax.experimental.pallas import tpu as pltpu
from jax.experimental.pallas import tpu_sc as plsc

N = 10080
E = 161280
ET = E + N
NTILES = 32
NPT = N // NTILES
EC = 64
EP = ((ET + EC - 1) // EC) * EC
GB = 8
NEG_SLOPE = 0.2
F32 = jnp.float32
I32 = jnp.int32


def _lanes(v):
    return jnp.zeros((16,), I32) + v


def _gat_edge_sc(h, asrc, adst, gmat, srcs, dsts, bounds, bias, *, C, H):
    CH = C // H
    NCH = CH // 16

    mesh = plsc.VectorSubcoreMesh(core_axis_name="c", subcore_axis_name="s")

    def body(h_hbm, asrc_hbm, adst_hbm, g_hbm, srcs_hbm, dsts_hbm, bounds_hbm,
             bias_hbm, out_hbm,
             asrc_v, adst_v, g_v, bias_v, bounds_v, srcs_buf, dsts_buf,
             rows, acc, stage, gs0, gs1, osem):
        wid = lax.axis_index("s") * 2 + lax.axis_index("c")
        n0 = wid * NPT

        pltpu.sync_copy(asrc_hbm, asrc_v)
        pltpu.sync_copy(adst_hbm, adst_v)
        pltpu.sync_copy(g_hbm, g_v)
        pltpu.sync_copy(bias_hbm, bias_v)
        pltpu.sync_copy(bounds_hbm, bounds_v)

        g = [g_v[hh] for hh in range(H)]
        zero16 = jnp.zeros((16,), F32)

        @plsc.parallel_loop(0, C // 16, unroll=8)
        def _(c2):
            acc[pl.ds(c2 * 16, 16)] = zero16

        e0 = jnp.max(plsc.load_gather(bounds_v, [_lanes(0) + wid]))
        e1 = jnp.max(plsc.load_gather(bounds_v, [_lanes(1) + wid]))

        pltpu.async_copy(stage, out_hbm.at[n0], osem)

        gsems = (gs0, gs1)

        def issue(bb, slot):
            pltpu.async_copy(
                h_hbm.at[srcs_buf.at[pl.ds(bb * GB, GB)]], rows.at[slot],
                gsems[slot])

        def flush(cn, den):
            pltpu.make_async_copy(stage, out_hbm.at[cn], osem).wait()
            for hh in range(H):
                rden = 1.0 / (den[hh] + 1e-16)

                @plsc.parallel_loop(hh * NCH, (hh + 1) * NCH, unroll=4)
                def _(c2, hh=hh, rden=rden):
                    sl = pl.ds(c2 * 16, 16)
                    stage[sl] = jnp.maximum(acc[sl] * rden + bias_v[sl], 0.0)
                    acc[sl] = zero16
            pltpu.async_copy(stage, out_hbm.at[cn], osem)

        def edge(le, slot, j, base, carry):
            e = base + le
            valid = jnp.logical_and(e >= e0, e < e1)

            def do(cn, *den):
                dst_bc = plsc.load_gather(dsts_buf, [_lanes(le)])
                dcur = jnp.max(dst_bc)

                def do_flush(cn, *den):
                    flush(cn, den)
                    return (dcur,) + tuple(zero16 for _ in range(H))

                def keep(cn, *den):
                    return (cn,) + tuple(den)

                cnd = lax.cond(dcur != cn, do_flush, keep, cn, *den)
                cn, den = cnd[0], list(cnd[1:])

                src_bc = plsc.load_gather(srcs_buf, [_lanes(le)])
                for hh in range(H):
                    as_h = plsc.load_gather(asrc_v, [src_bc * H + hh])
                    ad_h = plsc.load_gather(adst_v, [dst_bc * H + hh])
                    s = as_h + ad_h
                    al = jnp.where(s < 0.0, NEG_SLOPE * s, s)
                    t = g[hh] + ad_h
                    cb = jnp.where(t < 0.0, NEG_SLOPE * t, t)
                    ex = jnp.exp(al - cb)
                    den[hh] = den[hh] + ex

                    @plsc.parallel_loop(hh * NCH, (hh + 1) * NCH, unroll=8)
                    def _(c2, hh=hh, ex=ex, j=j, slot=slot):
                        sl = pl.ds(c2 * 16, 16)
                        plsc.addupdate(acc.at[sl], ex * rows[slot, j, sl])
                return (cn,) + tuple(den)

            def skip(cn, *den):
                return (cn,) + tuple(den)

            out = lax.cond(valid, do, skip, carry[0], *carry[1:])
            return (out[0],) + tuple(out[1:])

        def chunk(ci, carry):
            base = ci * EC
            pltpu.sync_copy(srcs_hbm.at[pl.ds(base, EC)], srcs_buf)
            pltpu.sync_copy(dsts_hbm.at[pl.ds(base, EC)], dsts_buf)
            issue(0, 0)
            for bb in range(EC // GB):
                slot = bb % 2
                if bb + 1 < EC // GB:
                    issue(bb + 1, (bb + 1) % 2)
                pltpu.make_async_copy(
                    h_hbm.at[srcs_buf.at[pl.ds(bb * GB, GB)]], rows.at[slot],
                    gsems[slot]).wait()

                def ebody(j, c, bb=bb, slot=slot):
                    return edge(bb * GB + j, slot, j, base, c)
                carry = lax.fori_loop(0, GB, ebody, carry)
            return carry

        c_lo = e0 // EC
        c_hi = (e1 + EC - 1) // EC
        init = (n0,) + tuple(jnp.zeros((16,), F32) for _ in range(H))
        fin = lax.fori_loop(c_lo, c_hi, chunk, init)
        flush(fin[0], list(fin[1:]))
        pltpu.make_async_copy(stage, out_hbm.at[n0], osem).wait()

    f = pl.kernel(
        body,
        out_type=jax.ShapeDtypeStruct((N, C), F32),
        mesh=mesh,
        compiler_params=pltpu.CompilerParams(needs_layout_passes=False),
        scratch_types=[
            pltpu.VMEM((N * H,), F32),
            pltpu.VMEM((N * H,), F32),
            pltpu.VMEM((H, 16), F32),
            pltpu.VMEM((C,), F32),
            pltpu.VMEM((48,), I32),
            pltpu.VMEM((EC,), I32),
            pltpu.VMEM((EC,), I32),
            pltpu.VMEM((2, GB, C), F32),
            pltpu.VMEM((C,), F32),
            pltpu.VMEM((C,), F32),
            pltpu.SemaphoreType.DMA,
            pltpu.SemaphoreType.DMA,
            pltpu.SemaphoreType.DMA,
        ],
    )
    return f(h, asrc, adst, gmat, srcs, dsts, bounds, bias)


def _prep_edges(edge_index):
    loops = jnp.arange(N, dtype=jnp.int32)
    src = jnp.concatenate([edge_index[0], loops])
    dst = jnp.concatenate([edge_index[1], loops])
    dst_s, src_s = jax.lax.sort_key_val(dst, src)
    tgt = jnp.arange(NTILES + 1, dtype=I32) * NPT
    bounds = jnp.searchsorted(dst_s, tgt, side="left").astype(I32)
    bounds = jnp.minimum(bounds, ET)
    bounds = jnp.concatenate([bounds, jnp.full((48 - NTILES - 1,), ET, I32)])
    src_p = jnp.concatenate([src_s, jnp.zeros((EP - ET,), I32)])
    dst_p = jnp.concatenate([dst_s, jnp.full((EP - ET,), N - 1, I32)])
    return src_p, dst_p, bounds


def _mlp_body(h_ref, fw1_ref, fb1_ref, fw2_ref, fb2_ref, fw3_ref, fb3_ref,
              fw4_ref, fb4_ref, o_ref):
    h = h_ref[...]
    p = jnp.sum(h.reshape(24, 420, 512), axis=1)
    p = jax.nn.relu(jnp.dot(p, fw1_ref[...], preferred_element_type=F32)
                    + fb1_ref[...])
    p = jax.nn.relu(jnp.dot(p, fw2_ref[...], preferred_element_type=F32)
                    + fb2_ref[...])
    p = jax.nn.relu(jnp.dot(p, fw3_ref[...], preferred_element_type=F32)
                    + fb3_ref[...])
    p = jnp.dot(p, fw4_ref[...], preferred_element_type=F32) + fb4_ref[...]
    o_ref[...] = p


def _pool_mlp(h, fw1, fb1, fw2, fb2, fw3, fb3, fw4, fb4):
    return pl.pallas_call(
        _mlp_body,
        out_shape=jax.ShapeDtypeStruct((24, 32), F32),
    )(h, fw1, fb1, fw2, fb2, fw3, fb3, fw4, fb4)


def _gat_layer(x, W, a_src, a_dst, b, srcs, dsts, bounds, *, heads, out_ch):
    h = x @ W
    hh = h.reshape(N, heads, out_ch)
    asrc = jnp.sum(hh * a_src, axis=-1)
    adst = jnp.sum(hh * a_dst, axis=-1)
    g = jnp.max(asrc, axis=0)
    gmat = jnp.broadcast_to(g[:, None], (heads, 16)).astype(F32)
    return _gat_edge_sc(h, asrc.reshape(-1), adst.reshape(-1), gmat,
                        srcs, dsts, bounds, b, C=heads * out_ch, H=heads)


def kernel(x, edge_index, W1, a1_src, a1_dst, b1, W2, a2_src, a2_dst, b2,
           fw1, fb1, fw2, fb2, fw3, fb3, fw4, fb4):
    srcs, dsts, bounds = _prep_edges(edge_index)
    h = _gat_layer(x, W1, a1_src, a1_dst, b1, srcs, dsts, bounds,
                   heads=2, out_ch=1024)
    h = _gat_layer(h, W2, a2_src, a2_dst, b2, srcs, dsts, bounds,
                   heads=1, out_ch=512)
    return _pool_mlp(h, fw1, fb1, fw2, fb2, fw3, fb3, fw4, fb4)


# speedup vs baseline: 12.2607x; 1.0099x over previous
import functools

import jax
import jax.numpy as jnp
from jax import lax
from jax.experimental import pallas as pl
from jax.experimental.pallas import tpu as pltpu
from jax.experimental.pallas import tpu_sc as plsc

N = 10080
E = 161280
ET = E + N
NTILES = 32
NPT = N // NTILES
EC = 64
EP = ((ET + EC - 1) // EC) * EC
GB = 16
NEG_SLOPE = 0.2
F32 = jnp.float32
I32 = jnp.int32


def _lanes(v):
    return jnp.zeros((16,), I32) + v


def _gat_edge_sc(h, asrc, adst, gmat, srcs, meta, bounds, bias, *, C, H):
    CH = C // H
    NCH = CH // 16

    mesh = plsc.VectorSubcoreMesh(core_axis_name="c", subcore_axis_name="s")

    def body(h_hbm, asrc_hbm, adst_hbm, g_hbm, srcs_hbm, meta_hbm, bounds_hbm,
             bias_hbm, out_hbm,
             asrc_v, adst_v, g_v, bias_v, bounds_v, srcs_buf, meta_buf,
             rows, acc, stage, gs0, gs1, osem):
        wid = lax.axis_index("s") * 2 + lax.axis_index("c")
        n0 = wid * NPT

        pltpu.sync_copy(asrc_hbm, asrc_v)
        pltpu.sync_copy(adst_hbm, adst_v)
        pltpu.sync_copy(g_hbm, g_v)
        pltpu.sync_copy(bias_hbm, bias_v)
        pltpu.sync_copy(bounds_hbm, bounds_v)

        g = [g_v[hh] for hh in range(H)]
        zero16 = jnp.zeros((16,), F32)

        @plsc.parallel_loop(0, C // 16, unroll=8)
        def _(c2):
            acc[pl.ds(c2 * 16, 16)] = zero16

        e0 = plsc.load_gather(bounds_v, [_lanes(0) + wid])[0]
        e1 = plsc.load_gather(bounds_v, [_lanes(1) + wid])[0]

        pltpu.async_copy(stage, out_hbm.at[n0], osem)

        gsems = (gs0, gs1)

        def issue(bb, slot):
            pltpu.async_copy(
                h_hbm.at[srcs_buf.at[pl.ds(bb * GB, GB)]], rows.at[slot],
                gsems[slot])

        def flush(cn, den):
            pltpu.make_async_copy(stage, out_hbm.at[cn], osem).wait()
            for hh in range(H):
                rden = 1.0 / (den[hh] + 1e-16)

                @plsc.parallel_loop(hh * NCH, (hh + 1) * NCH, unroll=4)
                def _(c2, hh=hh, rden=rden):
                    sl = pl.ds(c2 * 16, 16)
                    stage[sl] = jnp.maximum(acc[sl] * rden + bias_v[sl], 0.0)
                    acc[sl] = zero16
            pltpu.async_copy(stage, out_hbm.at[cn], osem)

        def edge(le, slot, j, base, carry, checked):
            def work(cn, *den):
                meta_bc = plsc.load_gather(meta_buf, [_lanes(le)])
                m0 = meta_bc[0]

                def do_flush(cn, *den):
                    flush(cn, den)
                    return ((m0 >> 1).astype(I32),) + tuple(
                        zero16 for _ in range(H))

                def keep(cn, *den):
                    return (cn,) + tuple(den)

                cnd = lax.cond((m0 & 1) == 1, do_flush, keep, cn, *den)
                cn, den = cnd[0], list(cnd[1:])

                src_bc = plsc.load_gather(srcs_buf, [_lanes(le)])
                dst_bc = lax.shift_right_logical(meta_bc, 1)
                for hh in range(H):
                    as_h = plsc.load_gather(asrc_v, [src_bc * H + hh])
                    ad_h = plsc.load_gather(adst_v, [dst_bc * H + hh])
                    s = as_h + ad_h
                    al = jnp.where(s < 0.0, NEG_SLOPE * s, s)
                    t = g[hh] + ad_h
                    cb = jnp.where(t < 0.0, NEG_SLOPE * t, t)
                    ex = jnp.exp(al - cb)
                    den[hh] = den[hh] + ex

                    @plsc.parallel_loop(hh * NCH, (hh + 1) * NCH, unroll=8)
                    def _(c2, hh=hh, ex=ex, j=j, slot=slot):
                        sl = pl.ds(c2 * 16, 16)
                        plsc.addupdate(acc.at[sl], ex * rows[slot, j, sl])
                return (cn,) + tuple(den)

            if not checked:
                return work(*carry)

            def skip(cn, *den):
                return (cn,) + tuple(den)

            e = base + le
            valid = jnp.logical_and(e >= e0, e < e1)
            return lax.cond(valid, work, skip, carry[0], *carry[1:])

        def make_chunk(checked):
            def chunk(ci, carry):
                base = ci * EC
                pltpu.sync_copy(srcs_hbm.at[pl.ds(base, EC)], srcs_buf)
                pltpu.sync_copy(meta_hbm.at[pl.ds(base, EC)], meta_buf)
                issue(0, 0)
                for bb in range(EC // GB):
                    slot = bb % 2
                    if bb + 1 < EC // GB:
                        issue(bb + 1, (bb + 1) % 2)
                    pltpu.make_async_copy(
                        h_hbm.at[srcs_buf.at[pl.ds(bb * GB, GB)]],
                        rows.at[slot], gsems[slot]).wait()

                    def ebody(j, c, bb=bb, slot=slot):
                        return edge(bb * GB + j, slot, j, base, c, checked)
                    carry = lax.fori_loop(0, GB, ebody, carry)
                return carry
            return chunk

        c_lo = e0 // EC
        c_hi = (e1 + EC - 1) // EC
        c_lo1 = jnp.minimum(c_lo + 1, c_hi)
        c_hm = jnp.maximum(c_hi - 1, c_lo1)
        init = (n0,) + tuple(jnp.zeros((16,), F32) for _ in range(H))
        carry = lax.fori_loop(c_lo, c_lo1, make_chunk(True), init)
        carry = lax.fori_loop(c_lo1, c_hm, make_chunk(False), carry)
        carry = lax.fori_loop(c_hm, c_hi, make_chunk(True), carry)
        flush(carry[0], list(carry[1:]))
        pltpu.make_async_copy(stage, out_hbm.at[n0], osem).wait()

    f = pl.kernel(
        body,
        out_type=jax.ShapeDtypeStruct((N, C), F32),
        mesh=mesh,
        compiler_params=pltpu.CompilerParams(needs_layout_passes=False),
        scratch_types=[
            pltpu.VMEM((N * H,), F32),
            pltpu.VMEM((N * H,), F32),
            pltpu.VMEM((H, 16), F32),
            pltpu.VMEM((C,), F32),
            pltpu.VMEM((48,), I32),
            pltpu.VMEM((EC,), I32),
            pltpu.VMEM((EC,), I32),
            pltpu.VMEM((2, GB, C), F32),
            pltpu.VMEM((C,), F32),
            pltpu.VMEM((C,), F32),
            pltpu.SemaphoreType.DMA,
            pltpu.SemaphoreType.DMA,
            pltpu.SemaphoreType.DMA,
        ],
    )
    return f(h, asrc, adst, gmat, srcs, meta, bounds, bias)


def _prep_edges(edge_index):
    loops = jnp.arange(N, dtype=jnp.int32)
    src = jnp.concatenate([edge_index[0], loops])
    dst = jnp.concatenate([edge_index[1], loops])
    dst_s, src_s = jax.lax.sort_key_val(dst, src)
    flag = jnp.concatenate([
        jnp.ones((1,), I32),
        (dst_s[1:] != dst_s[:-1]).astype(I32),
    ])
    meta = (dst_s << 1) | flag
    tgt = jnp.arange(NTILES + 1, dtype=I32) * NPT
    bounds = jnp.searchsorted(dst_s, tgt, side="left").astype(I32)
    bounds = jnp.minimum(bounds, ET)
    bounds = jnp.concatenate([bounds, jnp.full((48 - NTILES - 1,), ET, I32)])
    src_p = jnp.concatenate([src_s, jnp.zeros((EP - ET,), I32)])
    meta_p = jnp.concatenate([meta, jnp.full((EP - ET,), (N - 1) << 1, I32)])
    return src_p, meta_p, bounds


def _mlp_body(h_ref, fw1_ref, fb1_ref, fw2_ref, fb2_ref, fw3_ref, fb3_ref,
              fw4_ref, fb4_ref, o_ref):
    h = h_ref[...]
    p = jnp.sum(h.reshape(24, 420, 512), axis=1)
    p = jax.nn.relu(jnp.dot(p, fw1_ref[...], preferred_element_type=F32)
                    + fb1_ref[...])
    p = jax.nn.relu(jnp.dot(p, fw2_ref[...], preferred_element_type=F32)
                    + fb2_ref[...])
    p = jax.nn.relu(jnp.dot(p, fw3_ref[...], preferred_element_type=F32)
                    + fb3_ref[...])
    p = jnp.dot(p, fw4_ref[...], preferred_element_type=F32) + fb4_ref[...]
    o_ref[...] = p


def _pool_mlp(h, fw1, fb1, fw2, fb2, fw3, fb3, fw4, fb4):
    return pl.pallas_call(
        _mlp_body,
        out_shape=jax.ShapeDtypeStruct((24, 32), F32),
    )(h, fw1, fb1, fw2, fb2, fw3, fb3, fw4, fb4)


def _gat_layer(x, W, a_src, a_dst, b, srcs, meta, bounds, *, heads, out_ch):
    h = x @ W
    hh = h.reshape(N, heads, out_ch)
    asrc = jnp.sum(hh * a_src, axis=-1)
    adst = jnp.sum(hh * a_dst, axis=-1)
    g = jnp.max(asrc, axis=0)
    gmat = jnp.broadcast_to(g[:, None], (heads, 16)).astype(F32)
    return _gat_edge_sc(h, asrc.reshape(-1), adst.reshape(-1), gmat,
                        srcs, meta, bounds, b, C=heads * out_ch, H=heads)


def kernel(x, edge_index, W1, a1_src, a1_dst, b1, W2, a2_src, a2_dst, b2,
           fw1, fb1, fw2, fb2, fw3, fb3, fw4, fb4):
    srcs, meta, bounds = _prep_edges(edge_index)
    h = _gat_layer(x, W1, a1_src, a1_dst, b1, srcs, meta, bounds,
                   heads=2, out_ch=1024)
    h = _gat_layer(h, W2, a2_src, a2_dst, b2, srcs, meta, bounds,
                   heads=1, out_ch=512)
    return _pool_mlp(h, fw1, fb1, fw2, fb2, fw3, fb3, fw4, fb4)


# speedup vs baseline: 12.5092x; 1.0203x over previous
import functools

import jax
import jax.numpy as jnp
from jax import lax
from jax.experimental import pallas as pl
from jax.experimental.pallas import tpu as pltpu
from jax.experimental.pallas import tpu_sc as plsc

N = 10080
E = 161280
ET = E + N
NTILES = 32
NPT = N // NTILES
EC = 64
EP = ((ET + EC - 1) // EC) * EC
GB = 16
NEG_SLOPE = 0.2
F32 = jnp.float32
I32 = jnp.int32


def _lanes(v):
    return jnp.zeros((16,), I32) + v


def _gat_edge_sc(h, asrc, adst, gmat, srcs, meta, bounds, bias, *, C, H):
    CH = C // H
    NCH = CH // 16

    mesh = plsc.VectorSubcoreMesh(core_axis_name="c", subcore_axis_name="s")

    def body(h_hbm, asrc_hbm, adst_hbm, g_hbm, srcs_hbm, meta_hbm, bounds_hbm,
             bias_hbm, out_hbm,
             asrc_v, adst_v, g_v, bias_v, bounds_v, srcs_buf, meta_buf,
             rows, acc, stage, gs0, gs1, osem):
        wid = lax.axis_index("s") * 2 + lax.axis_index("c")
        n0 = wid * NPT

        pltpu.sync_copy(asrc_hbm, asrc_v)
        pltpu.sync_copy(adst_hbm, adst_v)
        pltpu.sync_copy(g_hbm, g_v)
        pltpu.sync_copy(bias_hbm, bias_v)
        pltpu.sync_copy(bounds_hbm, bounds_v)

        g = [g_v[hh] for hh in range(H)]
        zero16 = jnp.zeros((16,), F32)

        @plsc.parallel_loop(0, C // 16, unroll=8)
        def _(c2):
            acc[pl.ds(c2 * 16, 16)] = zero16

        e0 = plsc.load_gather(bounds_v, [_lanes(0) + wid])[0]
        e1 = plsc.load_gather(bounds_v, [_lanes(1) + wid])[0]

        pltpu.async_copy(stage, out_hbm.at[n0], osem)

        gsems = (gs0, gs1)

        def issue(bb, slot):
            pltpu.async_copy(
                h_hbm.at[srcs_buf.at[pl.ds(bb * GB, GB)]], rows.at[slot],
                gsems[slot])

        def flush(cn, den):
            pltpu.make_async_copy(stage, out_hbm.at[cn], osem).wait()
            for hh in range(H):
                rden = 1.0 / (den[hh] + 1e-16)

                @plsc.parallel_loop(hh * NCH, (hh + 1) * NCH, unroll=4)
                def _(c2, hh=hh, rden=rden):
                    sl = pl.ds(c2 * 16, 16)
                    stage[sl] = jnp.maximum(acc[sl] * rden + bias_v[sl], 0.0)
                    acc[sl] = zero16
            pltpu.async_copy(stage, out_hbm.at[cn], osem)

        def edge(le, slot, j, base, carry, checked):
            def work(cn, *den):
                meta_bc = plsc.load_gather(meta_buf, [_lanes(le)])
                m0 = meta_bc[0]

                def do_flush(cn, *den):
                    flush(cn, den)
                    return ((m0 >> 1).astype(I32),) + tuple(
                        zero16 for _ in range(H))

                def keep(cn, *den):
                    return (cn,) + tuple(den)

                cnd = lax.cond((m0 & 1) == 1, do_flush, keep, cn, *den)
                cn, den = cnd[0], list(cnd[1:])

                src_bc = plsc.load_gather(srcs_buf, [_lanes(le)])
                dst_bc = lax.shift_right_logical(meta_bc, 1)
                for hh in range(H):
                    as_h = plsc.load_gather(asrc_v, [src_bc * H + hh])
                    ad_h = plsc.load_gather(adst_v, [dst_bc * H + hh])
                    s = as_h + ad_h
                    al = jnp.where(s < 0.0, NEG_SLOPE * s, s)
                    t = g[hh] + ad_h
                    cb = jnp.where(t < 0.0, NEG_SLOPE * t, t)
                    ex = jnp.exp(al - cb)
                    den[hh] = den[hh] + ex

                    @plsc.parallel_loop(hh * NCH, (hh + 1) * NCH, unroll=8)
                    def _(c2, hh=hh, ex=ex, j=j, slot=slot):
                        sl = pl.ds(c2 * 16, 16)
                        acc[sl] = acc[sl] + ex * rows[slot, j, sl]
                return (cn,) + tuple(den)

            if not checked:
                return work(*carry)

            def skip(cn, *den):
                return (cn,) + tuple(den)

            e = base + le
            valid = jnp.logical_and(e >= e0, e < e1)
            return lax.cond(valid, work, skip, carry[0], *carry[1:])

        def make_chunk(checked):
            def chunk(ci, carry):
                base = ci * EC
                pltpu.sync_copy(srcs_hbm.at[pl.ds(base, EC)], srcs_buf)
                pltpu.sync_copy(meta_hbm.at[pl.ds(base, EC)], meta_buf)
                issue(0, 0)
                for bb in range(EC // GB):
                    slot = bb % 2
                    if bb + 1 < EC // GB:
                        issue(bb + 1, (bb + 1) % 2)
                    pltpu.make_async_copy(
                        h_hbm.at[srcs_buf.at[pl.ds(bb * GB, GB)]],
                        rows.at[slot], gsems[slot]).wait()

                    def ebody(j, c, bb=bb, slot=slot):
                        return edge(bb * GB + j, slot, j, base, c, checked)
                    carry = lax.fori_loop(0, GB, ebody, carry)
                return carry
            return chunk

        c_lo = e0 // EC
        c_hi = (e1 + EC - 1) // EC
        c_lo1 = jnp.minimum(c_lo + 1, c_hi)
        c_hm = jnp.maximum(c_hi - 1, c_lo1)
        init = (n0,) + tuple(jnp.zeros((16,), F32) for _ in range(H))
        carry = lax.fori_loop(c_lo, c_lo1, make_chunk(True), init)
        carry = lax.fori_loop(c_lo1, c_hm, make_chunk(False), carry)
        carry = lax.fori_loop(c_hm, c_hi, make_chunk(True), carry)
        flush(carry[0], list(carry[1:]))
        pltpu.make_async_copy(stage, out_hbm.at[n0], osem).wait()

    f = pl.kernel(
        body,
        out_type=jax.ShapeDtypeStruct((N, C), F32),
        mesh=mesh,
        compiler_params=pltpu.CompilerParams(needs_layout_passes=False),
        scratch_types=[
            pltpu.VMEM((N * H,), F32),
            pltpu.VMEM((N * H,), F32),
            pltpu.VMEM((H, 16), F32),
            pltpu.VMEM((C,), F32),
            pltpu.VMEM((48,), I32),
            pltpu.VMEM((EC,), I32),
            pltpu.VMEM((EC,), I32),
            pltpu.VMEM((2, GB, C), F32),
            pltpu.VMEM((C,), F32),
            pltpu.VMEM((C,), F32),
            pltpu.SemaphoreType.DMA,
            pltpu.SemaphoreType.DMA,
            pltpu.SemaphoreType.DMA,
        ],
    )
    return f(h, asrc, adst, gmat, srcs, meta, bounds, bias)


def _prep_edges(edge_index):
    loops = jnp.arange(N, dtype=jnp.int32)
    src = jnp.concatenate([edge_index[0], loops])
    dst = jnp.concatenate([edge_index[1], loops])
    dst_s, src_s = jax.lax.sort_key_val(dst, src)
    flag = jnp.concatenate([
        jnp.ones((1,), I32),
        (dst_s[1:] != dst_s[:-1]).astype(I32),
    ])
    meta = (dst_s << 1) | flag
    tgt = jnp.arange(NTILES + 1, dtype=I32) * NPT
    bounds = jnp.searchsorted(dst_s, tgt, side="left").astype(I32)
    bounds = jnp.minimum(bounds, ET)
    bounds = jnp.concatenate([bounds, jnp.full((48 - NTILES - 1,), ET, I32)])
    src_p = jnp.concatenate([src_s, jnp.zeros((EP - ET,), I32)])
    meta_p = jnp.concatenate([meta, jnp.full((EP - ET,), (N - 1) << 1, I32)])
    return src_p, meta_p, bounds


def _mlp_body(h_ref, fw1_ref, fb1_ref, fw2_ref, fb2_ref, fw3_ref, fb3_ref,
              fw4_ref, fb4_ref, o_ref):
    h = h_ref[...]
    p = jnp.sum(h.reshape(24, 420, 512), axis=1)
    p = jax.nn.relu(jnp.dot(p, fw1_ref[...], preferred_element_type=F32)
                    + fb1_ref[...])
    p = jax.nn.relu(jnp.dot(p, fw2_ref[...], preferred_element_type=F32)
                    + fb2_ref[...])
    p = jax.nn.relu(jnp.dot(p, fw3_ref[...], preferred_element_type=F32)
                    + fb3_ref[...])
    p = jnp.dot(p, fw4_ref[...], preferred_element_type=F32) + fb4_ref[...]
    o_ref[...] = p


def _pool_mlp(h, fw1, fb1, fw2, fb2, fw3, fb3, fw4, fb4):
    return pl.pallas_call(
        _mlp_body,
        out_shape=jax.ShapeDtypeStruct((24, 32), F32),
    )(h, fw1, fb1, fw2, fb2, fw3, fb3, fw4, fb4)


def _gat_layer(x, W, a_src, a_dst, b, srcs, meta, bounds, *, heads, out_ch):
    h = x @ W
    hh = h.reshape(N, heads, out_ch)
    asrc = jnp.sum(hh * a_src, axis=-1)
    adst = jnp.sum(hh * a_dst, axis=-1)
    g = jnp.max(asrc, axis=0)
    gmat = jnp.broadcast_to(g[:, None], (heads, 16)).astype(F32)
    return _gat_edge_sc(h, asrc.reshape(-1), adst.reshape(-1), gmat,
                        srcs, meta, bounds, b, C=heads * out_ch, H=heads)


def kernel(x, edge_index, W1, a1_src, a1_dst, b1, W2, a2_src, a2_dst, b2,
           fw1, fb1, fw2, fb2, fw3, fb3, fw4, fb4):
    srcs, meta, bounds = _prep_edges(edge_index)
    h = _gat_layer(x, W1, a1_src, a1_dst, b1, srcs, meta, bounds,
                   heads=2, out_ch=1024)
    h = _gat_layer(h, W2, a2_src, a2_dst, b2, srcs, meta, bounds,
                   heads=1, out_ch=512)
    return _pool_mlp(h, fw1, fb1, fw2, fb2, fw3, fb3, fw4, fb4)


# speedup vs baseline: 12.8631x; 1.0283x over previous
import functools

import jax
import jax.numpy as jnp
from jax import lax
from jax.experimental import pallas as pl
from jax.experimental.pallas import tpu as pltpu
from jax.experimental.pallas import tpu_sc as plsc

N = 10080
E = 161280
ET = E + N
NTILES = 32
NPT = N // NTILES
EC = 64
EP = ((ET + EC - 1) // EC) * EC
GB = 16
NEG_SLOPE = 0.2
F32 = jnp.float32
I32 = jnp.int32


def _lanes(v):
    return jnp.zeros((16,), I32) + v


def _gat_edge_sc(h, asrc, adst, gmat, srcs, meta, bounds, bias, *, C, H):
    CH = C // H
    NCH = CH // 16

    mesh = plsc.VectorSubcoreMesh(core_axis_name="c", subcore_axis_name="s")

    def body(h_hbm, asrc_hbm, adst_hbm, g_hbm, srcs_hbm, meta_hbm, bounds_hbm,
             bias_hbm, out_hbm,
             asrc_v, adst_v, g_v, bias_v, bounds_v, srcs_buf, meta_buf,
             rows, acc, stage, gs0, gs1, osem):
        wid = lax.axis_index("s") * 2 + lax.axis_index("c")
        n0 = wid * NPT

        pltpu.sync_copy(asrc_hbm, asrc_v)
        pltpu.sync_copy(adst_hbm, adst_v)
        pltpu.sync_copy(g_hbm, g_v)
        pltpu.sync_copy(bias_hbm, bias_v)
        pltpu.sync_copy(bounds_hbm, bounds_v)

        g = [g_v[hh] for hh in range(H)]
        zero16 = jnp.zeros((16,), F32)

        @plsc.parallel_loop(0, C // 16, unroll=8)
        def _(c2):
            acc[pl.ds(c2 * 16, 16)] = zero16

        e0 = plsc.load_gather(bounds_v, [_lanes(0) + wid])[0]
        e1 = plsc.load_gather(bounds_v, [_lanes(1) + wid])[0]

        pltpu.async_copy(stage, out_hbm.at[n0], osem)

        gsems = (gs0, gs1)

        def issue(bb, slot):
            pltpu.async_copy(
                h_hbm.at[srcs_buf.at[pl.ds(bb * GB, GB)]], rows.at[slot],
                gsems[slot])

        def flush(cn, den):
            pltpu.make_async_copy(stage, out_hbm.at[cn], osem).wait()
            for hh in range(H):
                rden = 1.0 / (den[hh] + 1e-16)

                @plsc.parallel_loop(hh * NCH, (hh + 1) * NCH, unroll=4)
                def _(c2, hh=hh, rden=rden):
                    sl = pl.ds(c2 * 16, 16)
                    stage[sl] = jnp.maximum(acc[sl] * rden + bias_v[sl], 0.0)
                    acc[sl] = zero16
            pltpu.async_copy(stage, out_hbm.at[cn], osem)

        def edge(le, slot, j, base, carry, checked):
            def work(cn, *den):
                meta_bc = plsc.load_gather(meta_buf, [_lanes(le)])
                m0 = meta_bc[0]

                def do_flush(cn, *den):
                    flush(cn, den)
                    return ((m0 >> 1).astype(I32),) + tuple(
                        zero16 for _ in range(H))

                def keep(cn, *den):
                    return (cn,) + tuple(den)

                cnd = lax.cond((m0 & 1) == 1, do_flush, keep, cn, *den)
                cn, den = cnd[0], list(cnd[1:])

                src_bc = plsc.load_gather(srcs_buf, [_lanes(le)])
                dst_bc = lax.shift_right_logical(meta_bc, 1)
                for hh in range(H):
                    as_h = plsc.load_gather(asrc_v, [src_bc + hh * N])
                    ad_h = plsc.load_gather(adst_v, [dst_bc + hh * N])
                    s = as_h + ad_h
                    al = jnp.where(s < 0.0, NEG_SLOPE * s, s)
                    t = g[hh] + ad_h
                    cb = jnp.where(t < 0.0, NEG_SLOPE * t, t)
                    ex = jnp.exp(al - cb)
                    den[hh] = den[hh] + ex

                    @plsc.parallel_loop(hh * NCH, (hh + 1) * NCH, unroll=8)
                    def _(c2, hh=hh, ex=ex, j=j, slot=slot):
                        sl = pl.ds(c2 * 16, 16)
                        acc[sl] = acc[sl] + ex * rows[slot, j, sl]
                return (cn,) + tuple(den)

            if not checked:
                return work(*carry)

            def skip(cn, *den):
                return (cn,) + tuple(den)

            e = base + le
            valid = jnp.logical_and(e >= e0, e < e1)
            return lax.cond(valid, work, skip, carry[0], *carry[1:])

        def make_chunk(checked):
            def chunk(ci, carry):
                base = ci * EC
                pltpu.sync_copy(srcs_hbm.at[pl.ds(base, EC)], srcs_buf)
                pltpu.sync_copy(meta_hbm.at[pl.ds(base, EC)], meta_buf)
                issue(0, 0)
                for bb in range(EC // GB):
                    slot = bb % 2
                    if bb + 1 < EC // GB:
                        issue(bb + 1, (bb + 1) % 2)
                    pltpu.make_async_copy(
                        h_hbm.at[srcs_buf.at[pl.ds(bb * GB, GB)]],
                        rows.at[slot], gsems[slot]).wait()

                    def ebody(j, c, bb=bb, slot=slot):
                        return edge(bb * GB + j, slot, j, base, c, checked)
                    carry = lax.fori_loop(0, GB, ebody, carry)
                return carry
            return chunk

        c_lo = e0 // EC
        c_hi = (e1 + EC - 1) // EC
        c_lo1 = jnp.minimum(c_lo + 1, c_hi)
        c_hm = jnp.maximum(c_hi - 1, c_lo1)
        init = (n0,) + tuple(jnp.zeros((16,), F32) for _ in range(H))
        carry = lax.fori_loop(c_lo, c_lo1, make_chunk(True), init)
        carry = lax.fori_loop(c_lo1, c_hm, make_chunk(False), carry)
        carry = lax.fori_loop(c_hm, c_hi, make_chunk(True), carry)
        flush(carry[0], list(carry[1:]))
        pltpu.make_async_copy(stage, out_hbm.at[n0], osem).wait()

    f = pl.kernel(
        body,
        out_type=jax.ShapeDtypeStruct((N, C), F32),
        mesh=mesh,
        compiler_params=pltpu.CompilerParams(needs_layout_passes=False),
        scratch_types=[
            pltpu.VMEM((N * H,), F32),
            pltpu.VMEM((N * H,), F32),
            pltpu.VMEM((H, 16), F32),
            pltpu.VMEM((C,), F32),
            pltpu.VMEM((48,), I32),
            pltpu.VMEM((EC,), I32),
            pltpu.VMEM((EC,), I32),
            pltpu.VMEM((2, GB, C), F32),
            pltpu.VMEM((C,), F32),
            pltpu.VMEM((C,), F32),
            pltpu.SemaphoreType.DMA,
            pltpu.SemaphoreType.DMA,
            pltpu.SemaphoreType.DMA,
        ],
    )
    return f(h, asrc, adst, gmat, srcs, meta, bounds, bias)


def _prep_edges(edge_index):
    loops = jnp.arange(N, dtype=jnp.int32)
    src = jnp.concatenate([edge_index[0], loops])
    dst = jnp.concatenate([edge_index[1], loops])
    dst_s, src_s = jax.lax.sort_key_val(dst, src)
    flag = jnp.concatenate([
        jnp.ones((1,), I32),
        (dst_s[1:] != dst_s[:-1]).astype(I32),
    ])
    meta = (dst_s << 1) | flag
    tgt = jnp.arange(NTILES + 1, dtype=I32) * NPT
    bounds = jnp.searchsorted(dst_s, tgt, side="left").astype(I32)
    bounds = jnp.minimum(bounds, ET)
    bounds = jnp.concatenate([bounds, jnp.full((48 - NTILES - 1,), ET, I32)])
    src_p = jnp.concatenate([src_s, jnp.zeros((EP - ET,), I32)])
    meta_p = jnp.concatenate([meta, jnp.full((EP - ET,), (N - 1) << 1, I32)])
    return src_p, meta_p, bounds


def _mlp_body(h_ref, fw1_ref, fb1_ref, fw2_ref, fb2_ref, fw3_ref, fb3_ref,
              fw4_ref, fb4_ref, o_ref):
    h = h_ref[...]
    p = jnp.sum(h.reshape(24, 420, 512), axis=1)
    p = jax.nn.relu(jnp.dot(p, fw1_ref[...], preferred_element_type=F32)
                    + fb1_ref[...])
    p = jax.nn.relu(jnp.dot(p, fw2_ref[...], preferred_element_type=F32)
                    + fb2_ref[...])
    p = jax.nn.relu(jnp.dot(p, fw3_ref[...], preferred_element_type=F32)
                    + fb3_ref[...])
    p = jnp.dot(p, fw4_ref[...], preferred_element_type=F32) + fb4_ref[...]
    o_ref[...] = p


def _pool_mlp(h, fw1, fb1, fw2, fb2, fw3, fb3, fw4, fb4):
    return pl.pallas_call(
        _mlp_body,
        out_shape=jax.ShapeDtypeStruct((24, 32), F32),
    )(h, fw1, fb1, fw2, fb2, fw3, fb3, fw4, fb4)


def _mm_alpha_body(x_ref, w_ref, as_ref, ad_ref, h_ref, asrc_ref, adst_ref):
    h = jnp.dot(x_ref[...], w_ref[...], preferred_element_type=F32)
    h_ref[...] = h
    asrc_ref[...] = jnp.sum(h * as_ref[0], axis=1, keepdims=True)[None]
    adst_ref[...] = jnp.sum(h * ad_ref[0], axis=1, keepdims=True)[None]


def _mm_alpha(x, W, a_src, a_dst, *, heads, out_ch):
    K = x.shape[1]
    BM = 360
    grid = (N // BM, heads)
    f = pl.pallas_call(
        _mm_alpha_body,
        grid=grid,
        in_specs=[
            pl.BlockSpec((BM, K), lambda mi, ni: (mi, 0)),
            pl.BlockSpec((K, out_ch), lambda mi, ni: (0, ni)),
            pl.BlockSpec((1, 1, out_ch), lambda mi, ni: (ni, 0, 0)),
            pl.BlockSpec((1, 1, out_ch), lambda mi, ni: (ni, 0, 0)),
        ],
        out_specs=[
            pl.BlockSpec((BM, out_ch), lambda mi, ni: (mi, ni)),
            pl.BlockSpec((1, BM, 1), lambda mi, ni: (ni, mi, 0)),
            pl.BlockSpec((1, BM, 1), lambda mi, ni: (ni, mi, 0)),
        ],
        out_shape=[
            jax.ShapeDtypeStruct((N, heads * out_ch), F32),
            jax.ShapeDtypeStruct((heads, N, 1), F32),
            jax.ShapeDtypeStruct((heads, N, 1), F32),
        ],
    )
    return f(x, W, a_src.reshape(heads, 1, out_ch),
             a_dst.reshape(heads, 1, out_ch))


def _gat_layer(x, W, a_src, a_dst, b, srcs, meta, bounds, *, heads, out_ch):
    h, asrc, adst = _mm_alpha(x, W, a_src, a_dst, heads=heads, out_ch=out_ch)
    g = jnp.max(asrc, axis=(1, 2))
    gmat = jnp.broadcast_to(g[:, None], (heads, 16)).astype(F32)
    return _gat_edge_sc(h, asrc.reshape(-1), adst.reshape(-1), gmat,
                        srcs, meta, bounds, b, C=heads * out_ch, H=heads)


def kernel(x, edge_index, W1, a1_src, a1_dst, b1, W2, a2_src, a2_dst, b2,
           fw1, fb1, fw2, fb2, fw3, fb3, fw4, fb4):
    srcs, meta, bounds = _prep_edges(edge_index)
    h = _gat_layer(x, W1, a1_src, a1_dst, b1, srcs, meta, bounds,
                   heads=2, out_ch=1024)
    h = _gat_layer(h, W2, a2_src, a2_dst, b2, srcs, meta, bounds,
                   heads=1, out_ch=512)
    return _pool_mlp(h, fw1, fb1, fw2, fb2, fw3, fb3, fw4, fb4)


# speedup vs baseline: 13.3951x; 1.0414x over previous
import functools

import jax
import jax.numpy as jnp
from jax import lax
from jax.experimental import pallas as pl
from jax.experimental.pallas import tpu as pltpu
from jax.experimental.pallas import tpu_sc as plsc

N = 10080
E = 161280
ET = E + N
NTILES = 32
NPT = N // NTILES
EC = 64
EP = ((ET + EC - 1) // EC) * EC
GB = 16
NEG_SLOPE = 0.2
F32 = jnp.float32
I32 = jnp.int32


def _lanes(v):
    return jnp.zeros((16,), I32) + v


def _gat_edge_sc(hp, asrc, adst, gmat, srcs, meta, bounds, bias, *, C, H):
    CH = C // H
    NCH = CH // 16
    NCB = CH // 32

    mesh = plsc.VectorSubcoreMesh(core_axis_name="c", subcore_axis_name="s")

    def body(h_hbm, asrc_hbm, adst_hbm, g_hbm, srcs_hbm, meta_hbm, bounds_hbm,
             bias_hbm, out_hbm,
             asrc_v, adst_v, g_v, bias_v, bounds_v, srcs_buf, meta_buf,
             rows, acc, stage, gs0, gs1, osem):
        wid = lax.axis_index("s") * 2 + lax.axis_index("c")
        n0 = wid * NPT

        pltpu.sync_copy(asrc_hbm, asrc_v)
        pltpu.sync_copy(adst_hbm, adst_v)
        pltpu.sync_copy(g_hbm, g_v)
        pltpu.sync_copy(bias_hbm, bias_v)
        pltpu.sync_copy(bounds_hbm, bounds_v)

        g = [g_v[hh] for hh in range(H)]
        zero16 = jnp.zeros((16,), F32)

        @plsc.parallel_loop(0, C // 16, unroll=8)
        def _(c2):
            acc[pl.ds(c2 * 16, 16)] = zero16

        e0 = plsc.load_gather(bounds_v, [_lanes(0) + wid])[0]
        e1 = plsc.load_gather(bounds_v, [_lanes(1) + wid])[0]

        pltpu.async_copy(stage, out_hbm.at[n0], osem)

        gsems = (gs0, gs1)

        def issue(bb, slot):
            pltpu.async_copy(
                h_hbm.at[srcs_buf.at[pl.ds(bb * GB, GB)]], rows.at[slot],
                gsems[slot])

        def flush(cn, den):
            pltpu.make_async_copy(stage, out_hbm.at[cn], osem).wait()
            for hh in range(H):
                rden = 1.0 / (den[hh] + 1e-16)

                @plsc.parallel_loop(hh * NCH, (hh + 1) * NCH, unroll=4)
                def _(c2, hh=hh, rden=rden):
                    sl = pl.ds(c2 * 16, 16)
                    stage[sl] = jnp.maximum(acc[sl] * rden + bias_v[sl], 0.0)
                    acc[sl] = zero16
            pltpu.async_copy(stage, out_hbm.at[cn], osem)

        def edge(le, slot, j, base, carry, checked):
            def work(cn, *den):
                meta_bc = plsc.load_gather(meta_buf, [_lanes(le)])
                m0 = meta_bc[0]

                def do_flush(cn, *den):
                    flush(cn, den)
                    return ((m0 >> 1).astype(I32),) + tuple(
                        zero16 for _ in range(H))

                def keep(cn, *den):
                    return (cn,) + tuple(den)

                cnd = lax.cond((m0 & 1) == 1, do_flush, keep, cn, *den)
                cn, den = cnd[0], list(cnd[1:])

                src_bc = plsc.load_gather(srcs_buf, [_lanes(le)])
                dst_bc = lax.shift_right_logical(meta_bc, 1)
                for hh in range(H):
                    as_h = plsc.load_gather(asrc_v, [src_bc + hh * N])
                    ad_h = plsc.load_gather(adst_v, [dst_bc + hh * N])
                    s = as_h + ad_h
                    al = jnp.where(s < 0.0, NEG_SLOPE * s, s)
                    t = g[hh] + ad_h
                    cb = jnp.where(t < 0.0, NEG_SLOPE * t, t)
                    ex = jnp.exp(al - cb)
                    den[hh] = den[hh] + ex

                    @plsc.parallel_loop(hh * NCB, (hh + 1) * NCB, unroll=4)
                    def _(c2, hh=hh, ex=ex, j=j, slot=slot):
                        xi = rows[slot, j, pl.ds(c2 * 16, 16)]
                        lo = plsc.bitcast(lax.shift_left(xi, 16), F32)
                        hi = plsc.bitcast(
                            lax.bitwise_and(xi, jnp.int32(-65536)), F32)
                        sl = pl.ds(c2 * 32, 16)
                        sh = pl.ds(c2 * 32 + 16, 16)
                        acc[sl] = acc[sl] + ex * lo
                        acc[sh] = acc[sh] + ex * hi
                return (cn,) + tuple(den)

            if not checked:
                return work(*carry)

            def skip(cn, *den):
                return (cn,) + tuple(den)

            e = base + le
            valid = jnp.logical_and(e >= e0, e < e1)
            return lax.cond(valid, work, skip, carry[0], *carry[1:])

        def make_chunk(checked):
            def chunk(ci, carry):
                base = ci * EC
                pltpu.sync_copy(srcs_hbm.at[pl.ds(base, EC)], srcs_buf)
                pltpu.sync_copy(meta_hbm.at[pl.ds(base, EC)], meta_buf)
                issue(0, 0)
                for bb in range(EC // GB):
                    slot = bb % 2
                    if bb + 1 < EC // GB:
                        issue(bb + 1, (bb + 1) % 2)
                    pltpu.make_async_copy(
                        h_hbm.at[srcs_buf.at[pl.ds(bb * GB, GB)]],
                        rows.at[slot], gsems[slot]).wait()

                    def ebody(j, c, bb=bb, slot=slot):
                        return edge(bb * GB + j, slot, j, base, c, checked)
                    carry = lax.fori_loop(0, GB, ebody, carry)
                return carry
            return chunk

        c_lo = e0 // EC
        c_hi = (e1 + EC - 1) // EC
        c_lo1 = jnp.minimum(c_lo + 1, c_hi)
        c_hm = jnp.maximum(c_hi - 1, c_lo1)
        init = (n0,) + tuple(jnp.zeros((16,), F32) for _ in range(H))
        carry = lax.fori_loop(c_lo, c_lo1, make_chunk(True), init)
        carry = lax.fori_loop(c_lo1, c_hm, make_chunk(False), carry)
        carry = lax.fori_loop(c_hm, c_hi, make_chunk(True), carry)
        flush(carry[0], list(carry[1:]))
        pltpu.make_async_copy(stage, out_hbm.at[n0], osem).wait()

    f = pl.kernel(
        body,
        out_type=jax.ShapeDtypeStruct((N, C), F32),
        mesh=mesh,
        compiler_params=pltpu.CompilerParams(needs_layout_passes=False),
        scratch_types=[
            pltpu.VMEM((N * H,), F32),
            pltpu.VMEM((N * H,), F32),
            pltpu.VMEM((H, 16), F32),
            pltpu.VMEM((C,), F32),
            pltpu.VMEM((48,), I32),
            pltpu.VMEM((EC,), I32),
            pltpu.VMEM((EC,), I32),
            pltpu.VMEM((2, GB, C // 2), I32),
            pltpu.VMEM((C,), F32),
            pltpu.VMEM((C,), F32),
            pltpu.SemaphoreType.DMA,
            pltpu.SemaphoreType.DMA,
            pltpu.SemaphoreType.DMA,
        ],
    )
    return f(hp, asrc, adst, gmat, srcs, meta, bounds, bias)


def _pack_rows(h, C):
    hb = h.astype(jnp.bfloat16).reshape(N, C // 32, 2, 16)
    hb = hb.transpose(0, 1, 3, 2).reshape(N, C // 2, 2)
    return lax.bitcast_convert_type(hb, I32)


def _prep_edges(edge_index):
    loops = jnp.arange(N, dtype=jnp.int32)
    src = jnp.concatenate([edge_index[0], loops])
    dst = jnp.concatenate([edge_index[1], loops])
    dst_s, src_s = jax.lax.sort_key_val(dst, src)
    flag = jnp.concatenate([
        jnp.ones((1,), I32),
        (dst_s[1:] != dst_s[:-1]).astype(I32),
    ])
    meta = (dst_s << 1) | flag
    tgt = jnp.arange(NTILES + 1, dtype=I32) * NPT
    bounds = jnp.searchsorted(dst_s, tgt, side="left").astype(I32)
    bounds = jnp.minimum(bounds, ET)
    bounds = jnp.concatenate([bounds, jnp.full((48 - NTILES - 1,), ET, I32)])
    src_p = jnp.concatenate([src_s, jnp.zeros((EP - ET,), I32)])
    meta_p = jnp.concatenate([meta, jnp.full((EP - ET,), (N - 1) << 1, I32)])
    return src_p, meta_p, bounds


def _mlp_body(h_ref, fw1_ref, fb1_ref, fw2_ref, fb2_ref, fw3_ref, fb3_ref,
              fw4_ref, fb4_ref, o_ref):
    h = h_ref[...]
    p = jnp.sum(h.reshape(24, 420, 512), axis=1)
    p = jax.nn.relu(jnp.dot(p, fw1_ref[...], preferred_element_type=F32)
                    + fb1_ref[...])
    p = jax.nn.relu(jnp.dot(p, fw2_ref[...], preferred_element_type=F32)
                    + fb2_ref[...])
    p = jax.nn.relu(jnp.dot(p, fw3_ref[...], preferred_element_type=F32)
                    + fb3_ref[...])
    p = jnp.dot(p, fw4_ref[...], preferred_element_type=F32) + fb4_ref[...]
    o_ref[...] = p


def _pool_mlp(h, fw1, fb1, fw2, fb2, fw3, fb3, fw4, fb4):
    return pl.pallas_call(
        _mlp_body,
        out_shape=jax.ShapeDtypeStruct((24, 32), F32),
    )(h, fw1, fb1, fw2, fb2, fw3, fb3, fw4, fb4)


def _mm_alpha_body(x_ref, w_ref, as_ref, ad_ref, h_ref, asrc_ref, adst_ref):
    h = jnp.dot(x_ref[...], w_ref[...], preferred_element_type=F32)
    h_ref[...] = h
    asrc_ref[...] = jnp.sum(h * as_ref[0], axis=1, keepdims=True)[None]
    adst_ref[...] = jnp.sum(h * ad_ref[0], axis=1, keepdims=True)[None]


def _mm_alpha(x, W, a_src, a_dst, *, heads, out_ch):
    K = x.shape[1]
    BM = 360
    grid = (N // BM, heads)
    f = pl.pallas_call(
        _mm_alpha_body,
        grid=grid,
        in_specs=[
            pl.BlockSpec((BM, K), lambda mi, ni: (mi, 0)),
            pl.BlockSpec((K, out_ch), lambda mi, ni: (0, ni)),
            pl.BlockSpec((1, 1, out_ch), lambda mi, ni: (ni, 0, 0)),
            pl.BlockSpec((1, 1, out_ch), lambda mi, ni: (ni, 0, 0)),
        ],
        out_specs=[
            pl.BlockSpec((BM, out_ch), lambda mi, ni: (mi, ni)),
            pl.BlockSpec((1, BM, 1), lambda mi, ni: (ni, mi, 0)),
            pl.BlockSpec((1, BM, 1), lambda mi, ni: (ni, mi, 0)),
        ],
        out_shape=[
            jax.ShapeDtypeStruct((N, heads * out_ch), F32),
            jax.ShapeDtypeStruct((heads, N, 1), F32),
            jax.ShapeDtypeStruct((heads, N, 1), F32),
        ],
    )
    return f(x, W, a_src.reshape(heads, 1, out_ch),
             a_dst.reshape(heads, 1, out_ch))


def _gat_layer(x, W, a_src, a_dst, b, srcs, meta, bounds, *, heads, out_ch):
    h, asrc, adst = _mm_alpha(x, W, a_src, a_dst, heads=heads, out_ch=out_ch)
    g = jnp.max(asrc, axis=(1, 2))
    gmat = jnp.broadcast_to(g[:, None], (heads, 16)).astype(F32)
    hp = _pack_rows(h, heads * out_ch)
    return _gat_edge_sc(hp, asrc.reshape(-1), adst.reshape(-1), gmat,
                        srcs, meta, bounds, b, C=heads * out_ch, H=heads)


def kernel(x, edge_index, W1, a1_src, a1_dst, b1, W2, a2_src, a2_dst, b2,
           fw1, fb1, fw2, fb2, fw3, fb3, fw4, fb4):
    srcs, meta, bounds = _prep_edges(edge_index)
    h = _gat_layer(x, W1, a1_src, a1_dst, b1, srcs, meta, bounds,
                   heads=2, out_ch=1024)
    h = _gat_layer(h, W2, a2_src, a2_dst, b2, srcs, meta, bounds,
                   heads=1, out_ch=512)
    return _pool_mlp(h, fw1, fb1, fw2, fb2, fw3, fb3, fw4, fb4)


# speedup vs baseline: 13.4013x; 1.0005x over previous
import jax
import jax.numpy as jnp
from jax import lax
from jax.experimental import pallas as pl
from jax.experimental.pallas import tpu as pltpu
from jax.experimental.pallas import tpu_sc as plsc

N = 10080
E = 161280
ET = E + N
NTILES = 32
NPT = N // NTILES
EC = 64
EP = ((ET + EC - 1) // EC) * EC
GB = 16
NEG_SLOPE = 0.2
F32 = jnp.float32
I32 = jnp.int32


def _lanes(v):
    return jnp.zeros((16,), I32) + v


def _gat_edge_sc(hp, asrc, adst, gmat, srcs, meta, bounds, bias, *, C, H):
    CH = C // H
    NCH = CH // 16
    NCB = CH // 32

    mesh = plsc.VectorSubcoreMesh(core_axis_name="c", subcore_axis_name="s")

    def body(h_hbm, asrc_hbm, adst_hbm, g_hbm, srcs_hbm, meta_hbm, bounds_hbm,
             bias_hbm, out_hbm,
             asrc_v, adst_v, g_v, bias_v, bounds_v, srcs_buf, meta_buf,
             rows, acc, stage, gs0, gs1, osem):
        wid = lax.axis_index("s") * 2 + lax.axis_index("c")
        n0 = wid * NPT

        pltpu.sync_copy(asrc_hbm, asrc_v)
        pltpu.sync_copy(adst_hbm, adst_v)
        pltpu.sync_copy(g_hbm, g_v)
        pltpu.sync_copy(bias_hbm, bias_v)
        pltpu.sync_copy(bounds_hbm, bounds_v)

        g = [g_v[hh] for hh in range(H)]
        zero16 = jnp.zeros((16,), F32)

        @plsc.parallel_loop(0, C // 16, unroll=8)
        def _(c2):
            acc[pl.ds(c2 * 16, 16)] = zero16

        e0 = plsc.load_gather(bounds_v, [_lanes(0) + wid])[0]
        e1 = plsc.load_gather(bounds_v, [_lanes(1) + wid])[0]

        pltpu.async_copy(stage, out_hbm.at[n0], osem)

        gsems = (gs0, gs1)

        def issue(bb, slot):
            pltpu.async_copy(
                h_hbm.at[srcs_buf.at[pl.ds(bb * GB, GB)]], rows.at[slot],
                gsems[slot])

        def flush(cn, den):
            pltpu.make_async_copy(stage, out_hbm.at[cn], osem).wait()
            for hh in range(H):
                rden = 1.0 / (den[hh] + 1e-16)

                @plsc.parallel_loop(hh * NCH, (hh + 1) * NCH, unroll=4)
                def _(c2, hh=hh, rden=rden):
                    sl = pl.ds(c2 * 16, 16)
                    stage[sl] = jnp.maximum(acc[sl] * rden + bias_v[sl], 0.0)
                    acc[sl] = zero16
            pltpu.async_copy(stage, out_hbm.at[cn], osem)

        def edge(le, slot, j, base, carry, checked):
            def work(cn, *den):
                meta_bc = plsc.load_gather(meta_buf, [_lanes(le)])
                m0 = meta_bc[0]

                def do_flush(cn, *den):
                    flush(cn, den)
                    return ((m0 >> 1).astype(I32),) + tuple(
                        zero16 for _ in range(H))

                def keep(cn, *den):
                    return (cn,) + tuple(den)

                cnd = lax.cond((m0 & 1) == 1, do_flush, keep, cn, *den)
                cn, den = cnd[0], list(cnd[1:])

                src_bc = plsc.load_gather(srcs_buf, [_lanes(le)])
                dst_bc = lax.shift_right_logical(meta_bc, 1)
                for hh in range(H):
                    as_h = plsc.load_gather(asrc_v, [src_bc + hh * N])
                    ad_h = plsc.load_gather(adst_v, [dst_bc + hh * N])
                    s = as_h + ad_h
                    al = jnp.where(s < 0.0, NEG_SLOPE * s, s)
                    t = g[hh] + ad_h
                    cb = jnp.where(t < 0.0, NEG_SLOPE * t, t)
                    ex = jnp.exp(al - cb)
                    den[hh] = den[hh] + ex

                    @plsc.parallel_loop(hh * NCB, (hh + 1) * NCB, unroll=4)
                    def _(c2, hh=hh, ex=ex, j=j, slot=slot):
                        xi = rows[slot, j, pl.ds(c2 * 16, 16)]
                        lo = plsc.bitcast(lax.shift_left(xi, 16), F32)
                        hi = plsc.bitcast(
                            lax.bitwise_and(xi, jnp.int32(-65536)), F32)
                        sl = pl.ds(c2 * 32, 16)
                        sh = pl.ds(c2 * 32 + 16, 16)
                        acc[sl] = acc[sl] + ex * lo
                        acc[sh] = acc[sh] + ex * hi
                return (cn,) + tuple(den)

            if not checked:
                return work(*carry)

            def skip(cn, *den):
                return (cn,) + tuple(den)

            e = base + le
            valid = jnp.logical_and(e >= e0, e < e1)
            return lax.cond(valid, work, skip, carry[0], *carry[1:])

        def make_chunk(checked):
            def chunk(ci, carry):
                base = ci * EC
                pltpu.sync_copy(srcs_hbm.at[pl.ds(base, EC)], srcs_buf)
                pltpu.sync_copy(meta_hbm.at[pl.ds(base, EC)], meta_buf)
                issue(0, 0)
                for bb in range(EC // GB):
                    slot = bb % 2
                    if bb + 1 < EC // GB:
                        issue(bb + 1, (bb + 1) % 2)
                    pltpu.make_async_copy(
                        h_hbm.at[srcs_buf.at[pl.ds(bb * GB, GB)]],
                        rows.at[slot], gsems[slot]).wait()

                    def ebody(j, c, bb=bb, slot=slot):
                        return edge(bb * GB + j, slot, j, base, c, checked)
                    carry = lax.fori_loop(0, GB, ebody, carry)
                return carry
            return chunk

        c_lo = e0 // EC
        c_hi = (e1 + EC - 1) // EC
        c_lo1 = jnp.minimum(c_lo + 1, c_hi)
        c_hm = jnp.maximum(c_hi - 1, c_lo1)
        init = (n0,) + tuple(jnp.zeros((16,), F32) for _ in range(H))
        carry = lax.fori_loop(c_lo, c_lo1, make_chunk(True), init)
        carry = lax.fori_loop(c_lo1, c_hm, make_chunk(False), carry)
        carry = lax.fori_loop(c_hm, c_hi, make_chunk(True), carry)
        flush(carry[0], list(carry[1:]))
        pltpu.make_async_copy(stage, out_hbm.at[n0], osem).wait()

    f = pl.kernel(
        body,
        out_type=jax.ShapeDtypeStruct((N, C), F32),
        mesh=mesh,
        compiler_params=pltpu.CompilerParams(needs_layout_passes=False),
        scratch_types=[
            pltpu.VMEM((N * H,), F32),
            pltpu.VMEM((N * H,), F32),
            pltpu.VMEM((H, 16), F32),
            pltpu.VMEM((C,), F32),
            pltpu.VMEM((48,), I32),
            pltpu.VMEM((EC,), I32),
            pltpu.VMEM((EC,), I32),
            pltpu.VMEM((2, GB, C // 2), I32),
            pltpu.VMEM((C,), F32),
            pltpu.VMEM((C,), F32),
            pltpu.SemaphoreType.DMA,
            pltpu.SemaphoreType.DMA,
            pltpu.SemaphoreType.DMA,
        ],
    )
    return f(hp, asrc, adst, gmat, srcs, meta, bounds, bias)


def _pack_rows(h, C):
    hb = h.astype(jnp.bfloat16).reshape(N, C // 32, 2, 16)
    hb = hb.transpose(0, 1, 3, 2).reshape(N, C // 2, 2)
    return lax.bitcast_convert_type(hb, I32)


def _prep_edges(edge_index):
    loops = jnp.arange(N, dtype=jnp.int32)
    src = jnp.concatenate([edge_index[0], loops])
    dst = jnp.concatenate([edge_index[1], loops])
    dst_s, src_s = jax.lax.sort_key_val(dst, src)
    flag = jnp.concatenate([
        jnp.ones((1,), I32),
        (dst_s[1:] != dst_s[:-1]).astype(I32),
    ])
    meta = (dst_s << 1) | flag
    tgt = jnp.arange(NTILES + 1, dtype=I32) * NPT
    bounds = jnp.searchsorted(dst_s, tgt, side="left").astype(I32)
    bounds = jnp.minimum(bounds, ET)
    bounds = jnp.concatenate([bounds, jnp.full((48 - NTILES - 1,), ET, I32)])
    src_p = jnp.concatenate([src_s, jnp.zeros((EP - ET,), I32)])
    meta_p = jnp.concatenate([meta, jnp.full((EP - ET,), (N - 1) << 1, I32)])
    return src_p, meta_p, bounds


def _mlp_body(h_ref, fw1_ref, fb1_ref, fw2_ref, fb2_ref, fw3_ref, fb3_ref,
              fw4_ref, fb4_ref, o_ref):
    h = h_ref[...]
    p = jnp.sum(h.reshape(24, 420, 512), axis=1)
    p = jax.nn.relu(jnp.dot(p, fw1_ref[...], preferred_element_type=F32)
                    + fb1_ref[...])
    p = jax.nn.relu(jnp.dot(p, fw2_ref[...], preferred_element_type=F32)
                    + fb2_ref[...])
    p = jax.nn.relu(jnp.dot(p, fw3_ref[...], preferred_element_type=F32)
                    + fb3_ref[...])
    p = jnp.dot(p, fw4_ref[...], preferred_element_type=F32) + fb4_ref[...]
    o_ref[...] = p


def _pool_mlp(h, fw1, fb1, fw2, fb2, fw3, fb3, fw4, fb4):
    return pl.pallas_call(
        _mlp_body,
        out_shape=jax.ShapeDtypeStruct((24, 32), F32),
    )(h, fw1, fb1, fw2, fb2, fw3, fb3, fw4, fb4)


def _mm_alpha_body(x_ref, w_ref, as_ref, ad_ref, h_ref, asrc_ref, adst_ref):
    h = jnp.dot(x_ref[...], w_ref[...], preferred_element_type=F32)
    h_ref[...] = h
    asrc_ref[...] = jnp.sum(h * as_ref[0], axis=1, keepdims=True)[None]
    adst_ref[...] = jnp.sum(h * ad_ref[0], axis=1, keepdims=True)[None]


def _mm_alpha(x, W, a_src, a_dst, *, heads, out_ch):
    K = x.shape[1]
    BM = 360
    grid = (N // BM, heads)
    f = pl.pallas_call(
        _mm_alpha_body,
        grid=grid,
        in_specs=[
            pl.BlockSpec((BM, K), lambda mi, ni: (mi, 0)),
            pl.BlockSpec((K, out_ch), lambda mi, ni: (0, ni)),
            pl.BlockSpec((1, 1, out_ch), lambda mi, ni: (ni, 0, 0)),
            pl.BlockSpec((1, 1, out_ch), lambda mi, ni: (ni, 0, 0)),
        ],
        out_specs=[
            pl.BlockSpec((BM, out_ch), lambda mi, ni: (mi, ni)),
            pl.BlockSpec((1, BM, 1), lambda mi, ni: (ni, mi, 0)),
            pl.BlockSpec((1, BM, 1), lambda mi, ni: (ni, mi, 0)),
        ],
        out_shape=[
            jax.ShapeDtypeStruct((N, heads * out_ch), F32),
            jax.ShapeDtypeStruct((heads, N, 1), F32),
            jax.ShapeDtypeStruct((heads, N, 1), F32),
        ],
    )
    return f(x, W, a_src.reshape(heads, 1, out_ch),
             a_dst.reshape(heads, 1, out_ch))


def _gat_layer(x, W, a_src, a_dst, b, srcs, meta, bounds, *, heads, out_ch):
    h, asrc, adst = _mm_alpha(x, W, a_src, a_dst, heads=heads, out_ch=out_ch)
    g = jnp.max(asrc, axis=(1, 2))
    gmat = jnp.broadcast_to(g[:, None], (heads, 16)).astype(F32)
    hp = _pack_rows(h, heads * out_ch)
    return _gat_edge_sc(hp, asrc.reshape(-1), adst.reshape(-1), gmat,
                        srcs, meta, bounds, b, C=heads * out_ch, H=heads)


def kernel(x, edge_index, W1, a1_src, a1_dst, b1, W2, a2_src, a2_dst, b2,
           fw1, fb1, fw2, fb2, fw3, fb3, fw4, fb4):
    srcs, meta, bounds = _prep_edges(edge_index)
    h = _gat_layer(x, W1, a1_src, a1_dst, b1, srcs, meta, bounds,
                   heads=2, out_ch=1024)
    h = _gat_layer(h, W2, a2_src, a2_dst, b2, srcs, meta, bounds,
                   heads=1, out_ch=512)
    return _pool_mlp(h, fw1, fb1, fw2, fb2, fw3, fb3, fw4, fb4)
